# baseline (speedup 1.0000x reference)
import functools
import jax
import jax.numpy as jnp
from jax import lax
from jax.experimental import pallas as pl
from jax.experimental.pallas import tpu as pltpu
from jax.experimental.pallas import tpu_sc as plsc


CHUNK_ROWS = 16


def kernel(tokens, W_pos):
    batch = tokens.shape[0]
    seq_len = tokens.shape[1]
    d = W_pos.shape[1]

    info = plsc.get_sparse_core_info()
    nw = info.num_cores * info.num_subcores
    rows_per_w = seq_len // nw
    nchunk = rows_per_w // CHUNK_ROWS

    mesh = plsc.VectorSubcoreMesh(core_axis_name="c", subcore_axis_name="s")

    @functools.partial(
        pl.kernel,
        mesh=mesh,
        out_type=jax.ShapeDtypeStruct((batch, seq_len, d), W_pos.dtype),
        scratch_types=[
            pltpu.VMEM((2, CHUNK_ROWS, d), W_pos.dtype),
            pltpu.SemaphoreType.DMA((2,)),
            pltpu.SemaphoreType.DMA((2,)),
        ],
    )
    def sc_copy(w_hbm, out_hbm, buf, rsem, wsem):
        wid = lax.axis_index("s") * info.num_cores + lax.axis_index("c")
        base = wid * rows_per_w

        def read(i):
            s = i % 2
            return pltpu.make_async_copy(
                w_hbm.at[pl.ds(base + i * CHUNK_ROWS, CHUNK_ROWS), :],
                buf.at[s], rsem.at[s])

        def write(i, b):
            s = i % 2
            return pltpu.make_async_copy(
                buf.at[s],
                out_hbm.at[b, pl.ds(base + i * CHUNK_ROWS, CHUNK_ROWS), :],
                wsem.at[s])

        read(0).start()
        for i in range(nchunk):
            read(i).wait()
            n = i + 1
            if n < nchunk:
                if i >= 1:
                    for b in range(batch):
                        write(i - 1, b).wait()
                read(n).start()
            for b in range(batch):
                write(i, b).start()
        for i in range(max(0, nchunk - 2), nchunk):
            for b in range(batch):
                write(i, b).wait()

    return sc_copy(W_pos[:seq_len])


# speedup vs baseline: 1.0197x; 1.0197x over previous
import functools
import jax
import jax.numpy as jnp
from jax import lax
from jax.experimental import pallas as pl
from jax.experimental.pallas import tpu as pltpu
from jax.experimental.pallas import tpu_sc as plsc


CHUNK_ROWS = 32


def kernel(tokens, W_pos):
    batch = tokens.shape[0]
    seq_len = tokens.shape[1]
    d = W_pos.shape[1]

    info = plsc.get_sparse_core_info()
    nw = info.num_cores * info.num_subcores
    rows_per_w = seq_len // nw
    nchunk = rows_per_w // CHUNK_ROWS

    mesh = plsc.VectorSubcoreMesh(core_axis_name="c", subcore_axis_name="s")

    @functools.partial(
        pl.kernel,
        mesh=mesh,
        out_type=jax.ShapeDtypeStruct((batch, seq_len, d), W_pos.dtype),
        scratch_types=[
            pltpu.VMEM((CHUNK_ROWS, d), W_pos.dtype),
            pltpu.SemaphoreType.DMA,
            pltpu.SemaphoreType.DMA,
        ],
    )
    def sc_copy(w_hbm, out_hbm, buf, rsem, wsem):
        wid = lax.axis_index("s") * info.num_cores + lax.axis_index("c")
        base = wid * rows_per_w

        def read(i):
            return pltpu.make_async_copy(
                w_hbm.at[pl.ds(base + i * CHUNK_ROWS, CHUNK_ROWS), :],
                buf, rsem)

        def write(i, b):
            return pltpu.make_async_copy(
                buf,
                out_hbm.at[b, pl.ds(base + i * CHUNK_ROWS, CHUNK_ROWS), :],
                wsem)

        for i in range(nchunk):
            read(i).start()
            read(i).wait()
            for b in range(batch):
                write(i, b).start()
            for b in range(batch):
                write(i, b).wait()

    return sc_copy(W_pos[:seq_len])


# speedup vs baseline: 1.3961x; 1.3691x over previous
import jax
import jax.numpy as jnp
from jax.experimental import pallas as pl
from jax.experimental.pallas import tpu as pltpu


ROWS = 1024
NBUF = 6
LEAD = 4


def _bcast_copy_kernel(w_hbm, out_hbm, buf, rsem, wsem):
    batch = out_hbm.shape[0]
    seq = w_hbm.shape[0]
    nchunk = seq // ROWS

    def read(c):
        s = c % NBUF
        return pltpu.make_async_copy(
            w_hbm.at[pl.ds(c * ROWS, ROWS), :], buf.at[s], rsem.at[s])

    def write(c, b):
        s = c % NBUF
        return pltpu.make_async_copy(
            buf.at[s], out_hbm.at[b, pl.ds(c * ROWS, ROWS), :], wsem.at[s])

    for c in range(min(LEAD, nchunk)):
        read(c).start()
    for c in range(nchunk):
        read(c).wait()
        for b in range(batch):
            write(c, b).start()
        n = c + LEAD
        if n < nchunk:
            prev = n - NBUF
            if prev >= 0:
                for b in range(batch):
                    write(prev, b).wait()
            read(n).start()
    for c in range(max(0, nchunk - NBUF), nchunk):
        for b in range(batch):
            write(c, b).wait()


def kernel(tokens, W_pos):
    batch = tokens.shape[0]
    seq_len = tokens.shape[1]
    d = W_pos.shape[1]

    out = pl.pallas_call(
        _bcast_copy_kernel,
        in_specs=[pl.BlockSpec(memory_space=pltpu.MemorySpace.HBM)],
        out_specs=pl.BlockSpec(memory_space=pltpu.MemorySpace.HBM),
        out_shape=jax.ShapeDtypeStruct((batch, seq_len, d), W_pos.dtype),
        scratch_shapes=[
            pltpu.VMEM((NBUF, ROWS, d), W_pos.dtype),
            pltpu.SemaphoreType.DMA((NBUF,)),
            pltpu.SemaphoreType.DMA((NBUF,)),
        ],
    )(W_pos[:seq_len])
    return out
